# baseline (speedup 1.0000x reference)
import functools

import jax
import jax.numpy as jnp
from jax import lax
from jax.experimental import pallas as pl
from jax.experimental.pallas import tpu as pltpu
from jax.experimental.pallas import tpu_sc as plsc

N = 10000
E = 320000
F_IN = 128
H = 256
C = 32
G = 64

NC = 2
NS = 16
K = 128
HH = H // 2
PAD = 128

def _round4(x):
    return -(-x // 4) * 4


EDGE_CHUNKS = _round4(-(-E // (NS * K)))
E_PAD = EDGE_CHUNKS * NS * K
NODE_CHUNKS = _round4(-(-N // (NS * K)))
N_PAD16 = NODE_CHUNKS * NS * K
EDGE_CHUNKS32 = _round4(-(-E // (NC * NS * K)))
E_PAD32 = EDGE_CHUNKS32 * NC * NS * K
NODE_CHUNKS32 = _round4(-(-N // (NC * NS * K)))
N_PAD32 = NODE_CHUNKS32 * NC * NS * K

BN = 1000


def _sc_mesh():
    return plsc.VectorSubcoreMesh(core_axis_name="c", subcore_axis_name="s")


def _copy_out_stripes(s, acc, out, c, n_out):
    if n_out % (NS * 8) == 0:
        rows = n_out // NS

        pltpu.sync_copy(acc.at[pl.ds(s * rows, rows)],
                        out.at[c].at[pl.ds(s * rows, rows)])
    elif n_out >= NS * 8:
        base = (n_out // NS) & ~7
        rem = n_out - base * NS

        @pl.when(s < NS - 1)
        def _():
            pltpu.sync_copy(acc.at[pl.ds(s * base, base)],
                            out.at[c].at[pl.ds(s * base, base)])

        @pl.when(s == NS - 1)
        def _():
            pltpu.sync_copy(acc.at[pl.ds((NS - 1) * base, base + rem)],
                            out.at[c].at[pl.ds((NS - 1) * base, base + rem)])
    else:
        @pl.when(s == 0)
        def _():
            pltpu.sync_copy(acc.at[pl.ds(0, n_out)], out.at[c])


@functools.cache
def _gs_pass(nc_chunks, n_tbl, n_acc, n_out):
    def body(tbl, src1, dst1, zeros, out, acc, idx_s, idx_d, rows):
        c = lax.axis_index("c")
        s = lax.axis_index("s")

        @pl.when(s == 0)
        def _():
            pltpu.sync_copy(zeros, acc)

        plsc.subcore_barrier()

        @pl.loop(0, nc_chunks)
        def _(i):
            base = (s * nc_chunks + i) * K
            pltpu.sync_copy(src1.at[pl.ds(base, K)], idx_s)
            pltpu.sync_copy(dst1.at[pl.ds(base, K)], idx_d)
            pltpu.sync_copy(tbl.at[c].at[idx_s], rows)
            pltpu.sync_copy(rows, acc.at[idx_d], add=True)

        plsc.subcore_barrier()
        _copy_out_stripes(s, acc, out, c, n_out)

    return pl.kernel(
        body,
        out_type=jax.ShapeDtypeStruct((NC, n_out, HH), jnp.float32),
        mesh=_sc_mesh(),
        scratch_types=[
            pltpu.VMEM_SHARED((n_acc, HH), jnp.float32),
            pltpu.VMEM((K,), jnp.int32),
            pltpu.VMEM((K,), jnp.int32),
            pltpu.VMEM((K, HH), jnp.float32),
        ],
    )


@functools.cache
def _gs_pipe(nc_chunks, n_tbl, n_acc, n_out):
    assert nc_chunks % 4 == 0

    def body(tbl, src1, dst1, zeros, out, acc, rows2, isr, idr, sem_z,
             sg0, sg1, ss0, ss1, sis0, sis1, sis2, sis3,
             sid0, sid1, sid2, sid3):
        sem_g = [sg0, sg1]
        sem_s = [ss0, ss1]
        sem_is = [sis0, sis1, sis2, sis3]
        sem_id = [sid0, sid1, sid2, sid3]
        c = lax.axis_index("c")
        s = lax.axis_index("s")

        @pl.when(s == 0)
        def _():
            pltpu.async_copy(zeros, acc, sem_z)

        def idx_start(n, j):
            base = (s * nc_chunks + n) * K
            pltpu.async_copy(src1.at[pl.ds(base, K)], isr.at[j], sem_is[j])
            pltpu.async_copy(dst1.at[pl.ds(base, K)], idr.at[j], sem_id[j])

        def idx_wait(j):
            pltpu.make_async_copy(src1.at[pl.ds(0, K)], isr.at[j],
                                  sem_is[j]).wait()
            pltpu.make_async_copy(dst1.at[pl.ds(0, K)], idr.at[j],
                                  sem_id[j]).wait()

        def g_desc(j, a):
            return pltpu.make_async_copy(tbl.at[c].at[isr.at[j]],
                                         rows2.at[a], sem_g[a])

        def s_desc(j, a):
            return pltpu.make_async_copy(rows2.at[a], acc.at[idr.at[j]],
                                         sem_s[a])

        idx_start(0, 0)
        idx_start(1, 1)

        @pl.when(s == 0)
        def _():
            pltpu.make_async_copy(zeros, acc, sem_z).wait()

        plsc.subcore_barrier()

        def stage(q, j):
            n = 4 * q + j
            a = j % 2

            @pl.when(n >= 2)
            def _():
                s_desc((j + 2) % 4, a).wait()

            idx_wait(j)
            g_desc(j, a).start()

            @pl.when(n + 2 < nc_chunks)
            def _():
                idx_start(n + 2, (j + 2) % 4)

            g_desc(j, a).wait()
            pltpu.async_copy(rows2.at[a], acc.at[idr.at[j]], sem_s[a],
                             add=True)

        @pl.loop(0, nc_chunks // 4)
        def _(q):
            for j in range(4):
                stage(q, j)

        s_desc(2, 0).wait()
        s_desc(3, 1).wait()
        plsc.subcore_barrier()
        _copy_out_stripes(s, acc, out, c, n_out)

    return pl.kernel(
        body,
        out_type=jax.ShapeDtypeStruct((NC, n_out, HH), jnp.float32),
        mesh=_sc_mesh(),
        scratch_types=[
            pltpu.VMEM_SHARED((n_acc, HH), jnp.float32),
            pltpu.VMEM((2, K, HH), jnp.float32),
            pltpu.VMEM((4, K), jnp.int32),
            pltpu.VMEM((4, K), jnp.int32),
        ] + [pltpu.SemaphoreType.DMA] * 13,
    )


@functools.cache
def _hist_pass(nc_chunks, n_acc, n_out):
    def body(dst1, ones_hbm, zeros, out, acc, idx_d, ones):
        c = lax.axis_index("c")
        s = lax.axis_index("s")

        pltpu.sync_copy(ones_hbm, ones)

        @pl.when(s == 0)
        def _():
            pltpu.sync_copy(zeros, acc)

        plsc.subcore_barrier()

        @pl.loop(0, nc_chunks)
        def _(i):
            base = ((c * NS + s) * nc_chunks + i) * K
            pltpu.sync_copy(dst1.at[pl.ds(base, K)], idx_d)
            pltpu.sync_copy(ones, acc.at[idx_d], add=True)

        plsc.subcore_barrier()
        _copy_out_stripes(s, acc, out, c, n_out)

    return pl.kernel(
        body,
        out_type=jax.ShapeDtypeStruct((NC, n_out, HH), jnp.float32),
        mesh=_sc_mesh(),
        scratch_types=[
            pltpu.VMEM_SHARED((n_acc, HH), jnp.float32),
            pltpu.VMEM((K,), jnp.int32),
            pltpu.VMEM((K, HH), jnp.float32),
        ],
    )


@functools.cache
def _hist_pipe(nc_chunks, n_acc, n_out):
    assert nc_chunks % 4 == 0

    def body(dst1, ones_hbm, zeros, out, acc, ones, idr,
             sem_z, sem_o, ss0, ss1, ss2, ss3, si0, si1, si2, si3):
        sem_s = [ss0, ss1, ss2, ss3]
        sem_i = [si0, si1, si2, si3]
        c = lax.axis_index("c")
        s = lax.axis_index("s")

        @pl.when(s == 0)
        def _():
            pltpu.async_copy(zeros, acc, sem_z)

        pltpu.async_copy(ones_hbm, ones, sem_o)

        def idx_start(n, j):
            base = ((c * NS + s) * nc_chunks + n) * K
            pltpu.async_copy(dst1.at[pl.ds(base, K)], idr.at[j], sem_i[j])

        def idx_wait(j):
            pltpu.make_async_copy(dst1.at[pl.ds(0, K)], idr.at[j],
                                  sem_i[j]).wait()

        def s_desc(j):
            return pltpu.make_async_copy(ones, acc.at[idr.at[j]], sem_s[j])

        idx_start(0, 0)
        idx_start(1, 1)
        pltpu.make_async_copy(ones_hbm, ones, sem_o).wait()

        @pl.when(s == 0)
        def _():
            pltpu.make_async_copy(zeros, acc, sem_z).wait()

        plsc.subcore_barrier()

        def stage(q, j):
            n = 4 * q + j

            @pl.when(n >= 2)
            def _():
                s_desc((j + 2) % 4).wait()

            idx_wait(j)

            @pl.when(n + 2 < nc_chunks)
            def _():
                idx_start(n + 2, (j + 2) % 4)

            pltpu.async_copy(ones, acc.at[idr.at[j]], sem_s[j], add=True)

        @pl.loop(0, nc_chunks // 4)
        def _(q):
            for j in range(4):
                stage(q, j)

        s_desc(2).wait()
        s_desc(3).wait()
        plsc.subcore_barrier()
        _copy_out_stripes(s, acc, out, c, n_out)

    return pl.kernel(
        body,
        out_type=jax.ShapeDtypeStruct((NC, n_out, HH), jnp.float32),
        mesh=_sc_mesh(),
        scratch_types=[
            pltpu.VMEM_SHARED((n_acc, HH), jnp.float32),
            pltpu.VMEM((K, HH), jnp.float32),
            pltpu.VMEM((4, K), jnp.int32),
        ] + [pltpu.SemaphoreType.DMA] * 10,
    )


def _t_pre(x, W1, deg2):
    def body(x_ref, w_ref, d_ref, zs_ref, inv_ref):
        deg = d_ref[0, :, 0:1] + d_ref[1, :, 0:1] + 1.0
        inv = lax.rsqrt(deg)
        z = jnp.dot(x_ref[...], w_ref[...], preferred_element_type=jnp.float32)
        zs = z * inv
        zs_ref[0] = zs[:, :HH]
        zs_ref[1] = zs[:, HH:]
        inv_ref[...] = inv

    return pl.pallas_call(
        body,
        grid=(N // BN,),
        in_specs=[
            pl.BlockSpec((BN, F_IN), lambda i: (i, 0)),
            pl.BlockSpec((F_IN, H), lambda i: (0, 0)),
            pl.BlockSpec((NC, BN, HH), lambda i: (0, i, 0)),
        ],
        out_specs=[
            pl.BlockSpec((NC, BN, HH), lambda i: (0, i, 0)),
            pl.BlockSpec((BN, 1), lambda i: (i, 0)),
        ],
        out_shape=[
            jax.ShapeDtypeStruct((NC, N, HH), jnp.float32),
            jax.ShapeDtypeStruct((N, 1), jnp.float32),
        ],
    )(x, W1, deg2)


def _t_mid(s2, zs2, inv, b_row, W):
    def body(s_ref, zs_ref, inv_ref, b_ref, w_ref, o_ref):
        agg = jnp.concatenate(
            [s_ref[0] + zs_ref[0], s_ref[1] + zs_ref[1]], axis=1)
        inv = inv_ref[...]
        h = agg * inv + b_ref[...]
        h = jnp.where(h > 0, h, 0.01 * h)
        z = jnp.dot(h, w_ref[...], preferred_element_type=jnp.float32)
        zn = z * inv
        o_ref[0] = zn[:, :HH]
        o_ref[1] = zn[:, HH:]

    return pl.pallas_call(
        body,
        grid=(N // BN,),
        in_specs=[
            pl.BlockSpec((NC, BN, HH), lambda i: (0, i, 0)),
            pl.BlockSpec((NC, BN, HH), lambda i: (0, i, 0)),
            pl.BlockSpec((BN, 1), lambda i: (i, 0)),
            pl.BlockSpec((1, H), lambda i: (0, 0)),
            pl.BlockSpec((H, H), lambda i: (0, 0)),
        ],
        out_specs=pl.BlockSpec((NC, BN, HH), lambda i: (0, i, 0)),
        out_shape=jax.ShapeDtypeStruct((NC, N, HH), jnp.float32),
    )(s2, zs2, inv, b_row, W)


def _t_post(s3, zs3, inv, b_row):
    def body(s_ref, zs_ref, inv_ref, b_ref, o_ref):
        agg = jnp.concatenate(
            [s_ref[0] + zs_ref[0], s_ref[1] + zs_ref[1]], axis=1)
        h = agg * inv_ref[...] + b_ref[...]
        o_ref[0] = h[:, :HH]
        o_ref[1] = h[:, HH:]

    return pl.pallas_call(
        body,
        grid=(N // BN,),
        in_specs=[
            pl.BlockSpec((NC, BN, HH), lambda i: (0, i, 0)),
            pl.BlockSpec((NC, BN, HH), lambda i: (0, i, 0)),
            pl.BlockSpec((BN, 1), lambda i: (i, 0)),
            pl.BlockSpec((1, H), lambda i: (0, 0)),
        ],
        out_specs=pl.BlockSpec((NC, BN, HH), lambda i: (0, i, 0)),
        out_shape=jax.ShapeDtypeStruct((NC, N, HH), jnp.float32),
    )(s3, zs3, inv, b_row)


def _t_head(psum, cnt, Wlin, blin_row, Wout, bout_row):
    def body(ps_ref, c_ref, wl_ref, bl_ref, wo_ref, bo_ref,
             logits_ref, probs_ref, emb_ref, el_ref):
        sum_pool = jnp.concatenate([ps_ref[0], ps_ref[1]], axis=1)
        counts = c_ref[0, :, 0:1] + c_ref[1, :, 0:1]
        mean_pool = sum_pool / jnp.maximum(counts, 1.0)
        embeds = jnp.concatenate([sum_pool, mean_pool], axis=1)
        hl = jnp.dot(embeds, wl_ref[...],
                     preferred_element_type=jnp.float32) + bl_ref[...]
        el = jnp.maximum(hl, 0.0)
        logits = jnp.dot(el, wo_ref[...],
                         preferred_element_type=jnp.float32) + bo_ref[...]
        m = jnp.max(logits, axis=1, keepdims=True)
        ex = jnp.exp(logits - m)
        probs = ex / jnp.sum(ex, axis=1, keepdims=True)
        logits_ref[...] = logits
        probs_ref[...] = probs
        emb_ref[...] = embeds
        el_ref[...] = el

    return pl.pallas_call(
        body,
        out_shape=[
            jax.ShapeDtypeStruct((G, C), jnp.float32),
            jax.ShapeDtypeStruct((G, C), jnp.float32),
            jax.ShapeDtypeStruct((G, 2 * H), jnp.float32),
            jax.ShapeDtypeStruct((G, H), jnp.float32),
        ],
    )(psum, cnt, Wlin, blin_row, Wout, bout_row)


def _pad_reshape(idx, total, fill, shape):
    pad = jnp.full((total - idx.shape[0],), fill, jnp.int32)
    return jnp.concatenate([idx, pad]).reshape(shape)


def kernel(x, edge_index, batch, W1, b1, W2, b2, W3, b3, Wlin, blin, Wout, bout):
    src = edge_index[0]
    dst = edge_index[1]

    src_e = _pad_reshape(src, E_PAD, 0, (E_PAD,))
    dst_e = _pad_reshape(dst, E_PAD, N, (E_PAD,))
    dst_h = _pad_reshape(dst, E_PAD32, N, (E_PAD32,))
    node_p = _pad_reshape(jnp.arange(N, dtype=jnp.int32), N_PAD16, 0,
                          (N_PAD16,))
    batch_p = _pad_reshape(batch, N_PAD16, G, (N_PAD16,))
    batch_h = _pad_reshape(batch, N_PAD32, G, (N_PAD32,))

    z_nodes = jnp.zeros((N + PAD, HH), jnp.float32)
    z_pool = jnp.zeros((G + PAD, HH), jnp.float32)

    gs_edge = _gs_pipe(EDGE_CHUNKS, N, N + PAD, N)
    gs_pool = _gs_pipe(NODE_CHUNKS, N, G + PAD, G)
    hist_deg = _hist_pipe(EDGE_CHUNKS32, N + PAD, N)
    hist_cnt = _hist_pipe(NODE_CHUNKS32, G + PAD, G)

    ones_k = jnp.ones((K, HH), jnp.float32)

    deg2 = hist_deg(dst_h, ones_k, z_nodes)
    zs1, inv = _t_pre(x, W1, deg2)
    s1 = gs_edge(zs1, src_e, dst_e, z_nodes)
    zs2 = _t_mid(s1, zs1, inv, b1.reshape(1, H), W2)
    s2 = gs_edge(zs2, src_e, dst_e, z_nodes)
    zs3 = _t_mid(s2, zs2, inv, b2.reshape(1, H), W3)
    s3 = gs_edge(zs3, src_e, dst_e, z_nodes)
    h3 = _t_post(s3, zs3, inv, b3.reshape(1, H))
    psum = gs_pool(h3, node_p, batch_p, z_pool)
    z_cnt = z_pool + psum[0, 0, 0] * 0.0
    cnt = hist_cnt(batch_h, ones_k, z_cnt)
    logits, probs, embeds, embeds_last = _t_head(
        psum, cnt, Wlin, blin.reshape(1, H), Wout, bout.reshape(1, C))
    return (logits, probs, embeds, embeds_last)


# speedup vs baseline: 1.6524x; 1.6524x over previous
import functools

import jax
import jax.numpy as jnp
from jax import lax
from jax.experimental import pallas as pl
from jax.experimental.pallas import tpu as pltpu
from jax.experimental.pallas import tpu_sc as plsc

N = 10000
E = 320000
F_IN = 128
H = 256
C = 32
G = 64

NC = 2
NS = 16
K = 128
HH = H // 2
PAD = 128

IB = 8


def _round8(x):
    return -(-x // IB) * IB


EDGE_CHUNKS = _round8(-(-E // (NS * K)))
E_PAD = EDGE_CHUNKS * NS * K
NODE_CHUNKS = _round8(-(-N // (NS * K)))
N_PAD16 = NODE_CHUNKS * NS * K
EDGE_CHUNKS32 = _round8(-(-E // (NC * NS * K)))
E_PAD32 = EDGE_CHUNKS32 * NC * NS * K
NODE_CHUNKS32 = _round8(-(-N // (NC * NS * K)))
N_PAD32 = NODE_CHUNKS32 * NC * NS * K

BN = 1000


def _sc_mesh():
    return plsc.VectorSubcoreMesh(core_axis_name="c", subcore_axis_name="s")


def _copy_out_stripes(s, acc, out, c, n_out):
    if n_out % (NS * 8) == 0:
        rows = n_out // NS

        pltpu.sync_copy(acc.at[pl.ds(s * rows, rows)],
                        out.at[c].at[pl.ds(s * rows, rows)])
    elif n_out >= NS * 8:
        base = (n_out // NS) & ~7
        rem = n_out - base * NS

        @pl.when(s < NS - 1)
        def _():
            pltpu.sync_copy(acc.at[pl.ds(s * base, base)],
                            out.at[c].at[pl.ds(s * base, base)])

        @pl.when(s == NS - 1)
        def _():
            pltpu.sync_copy(acc.at[pl.ds((NS - 1) * base, base + rem)],
                            out.at[c].at[pl.ds((NS - 1) * base, base + rem)])
    else:
        @pl.when(s == 0)
        def _():
            pltpu.sync_copy(acc.at[pl.ds(0, n_out)], out.at[c])


@functools.cache
def _gs_pass(nc_chunks, n_tbl, n_acc, n_out):
    assert nc_chunks % IB == 0

    def body(tbl, src3, dst3, zeros, out, acc, idx_s, idx_d, rows):
        c = lax.axis_index("c")
        s = lax.axis_index("s")

        @pl.when(s == 0)
        def _():
            pltpu.sync_copy(zeros, acc)

        plsc.subcore_barrier()

        @pl.loop(0, nc_chunks // IB)
        def _(b):
            pltpu.sync_copy(src3.at[s, pl.ds(b * IB, IB)], idx_s)
            pltpu.sync_copy(dst3.at[s, pl.ds(b * IB, IB)], idx_d)
            for j in range(IB):
                pltpu.sync_copy(tbl.at[c].at[idx_s.at[j]], rows)
                pltpu.sync_copy(rows, acc.at[idx_d.at[j]], add=True)

        plsc.subcore_barrier()
        _copy_out_stripes(s, acc, out, c, n_out)

    return pl.kernel(
        body,
        out_type=jax.ShapeDtypeStruct((NC, n_out, HH), jnp.float32),
        mesh=_sc_mesh(),
        scratch_types=[
            pltpu.VMEM_SHARED((n_acc, HH), jnp.float32),
            pltpu.VMEM((IB, K), jnp.int32),
            pltpu.VMEM((IB, K), jnp.int32),
            pltpu.VMEM((K, HH), jnp.float32),
        ],
    )


@functools.cache
def _gs_pipe(nc_chunks, n_tbl, n_acc, n_out):
    assert nc_chunks % 4 == 0

    def body(tbl, src1, dst1, zeros, out, acc, rows2, isr, idr, sem_z,
             sg0, sg1, ss0, ss1, sis0, sis1, sis2, sis3,
             sid0, sid1, sid2, sid3):
        sem_g = [sg0, sg1]
        sem_s = [ss0, ss1]
        sem_is = [sis0, sis1, sis2, sis3]
        sem_id = [sid0, sid1, sid2, sid3]
        c = lax.axis_index("c")
        s = lax.axis_index("s")

        @pl.when(s == 0)
        def _():
            pltpu.async_copy(zeros, acc, sem_z)

        def idx_start(n, j):
            base = (s * nc_chunks + n) * K
            pltpu.async_copy(src1.at[pl.ds(base, K)], isr.at[j], sem_is[j])
            pltpu.async_copy(dst1.at[pl.ds(base, K)], idr.at[j], sem_id[j])

        def idx_wait(j):
            pltpu.make_async_copy(src1.at[pl.ds(0, K)], isr.at[j],
                                  sem_is[j]).wait()
            pltpu.make_async_copy(dst1.at[pl.ds(0, K)], idr.at[j],
                                  sem_id[j]).wait()

        def g_desc(j, a):
            return pltpu.make_async_copy(tbl.at[c].at[isr.at[j]],
                                         rows2.at[a], sem_g[a])

        def s_desc(j, a):
            return pltpu.make_async_copy(rows2.at[a], acc.at[idr.at[j]],
                                         sem_s[a])

        idx_start(0, 0)
        idx_start(1, 1)

        @pl.when(s == 0)
        def _():
            pltpu.make_async_copy(zeros, acc, sem_z).wait()

        plsc.subcore_barrier()

        def stage(q, j):
            n = 4 * q + j
            a = j % 2

            @pl.when(n >= 2)
            def _():
                s_desc((j + 2) % 4, a).wait()

            idx_wait(j)
            g_desc(j, a).start()

            @pl.when(n + 2 < nc_chunks)
            def _():
                idx_start(n + 2, (j + 2) % 4)

            g_desc(j, a).wait()
            pltpu.async_copy(rows2.at[a], acc.at[idr.at[j]], sem_s[a],
                             add=True)

        @pl.loop(0, nc_chunks // 4)
        def _(q):
            for j in range(4):
                stage(q, j)

        s_desc(2, 0).wait()
        s_desc(3, 1).wait()
        plsc.subcore_barrier()
        _copy_out_stripes(s, acc, out, c, n_out)

    return pl.kernel(
        body,
        out_type=jax.ShapeDtypeStruct((NC, n_out, HH), jnp.float32),
        mesh=_sc_mesh(),
        scratch_types=[
            pltpu.VMEM_SHARED((n_acc, HH), jnp.float32),
            pltpu.VMEM((2, K, HH), jnp.float32),
            pltpu.VMEM((4, K), jnp.int32),
            pltpu.VMEM((4, K), jnp.int32),
        ] + [pltpu.SemaphoreType.DMA] * 13,
    )


@functools.cache
def _hist_pass(nc_chunks, n_acc, n_out):
    assert nc_chunks % IB == 0

    def body(dst4, ones_hbm, zeros, out, acc, idx_d, ones):
        c = lax.axis_index("c")
        s = lax.axis_index("s")

        pltpu.sync_copy(ones_hbm, ones)

        @pl.when(s == 0)
        def _():
            pltpu.sync_copy(zeros, acc)

        plsc.subcore_barrier()

        @pl.loop(0, nc_chunks // IB)
        def _(b):
            pltpu.sync_copy(dst4.at[c, s, pl.ds(b * IB, IB)], idx_d)
            for j in range(IB):
                pltpu.sync_copy(ones, acc.at[idx_d.at[j]], add=True)

        plsc.subcore_barrier()
        _copy_out_stripes(s, acc, out, c, n_out)

    return pl.kernel(
        body,
        out_type=jax.ShapeDtypeStruct((NC, n_out, HH), jnp.float32),
        mesh=_sc_mesh(),
        scratch_types=[
            pltpu.VMEM_SHARED((n_acc, HH), jnp.float32),
            pltpu.VMEM((IB, K), jnp.int32),
            pltpu.VMEM((K, HH), jnp.float32),
        ],
    )


@functools.cache
def _hist_pipe(nc_chunks, n_acc, n_out):
    assert nc_chunks % 4 == 0

    def body(dst1, ones_hbm, zeros, out, acc, ones, idr,
             sem_z, sem_o, ss0, ss1, ss2, ss3, si0, si1, si2, si3):
        sem_s = [ss0, ss1, ss2, ss3]
        sem_i = [si0, si1, si2, si3]
        c = lax.axis_index("c")
        s = lax.axis_index("s")

        @pl.when(s == 0)
        def _():
            pltpu.async_copy(zeros, acc, sem_z)

        pltpu.async_copy(ones_hbm, ones, sem_o)

        def idx_start(n, j):
            base = ((c * NS + s) * nc_chunks + n) * K
            pltpu.async_copy(dst1.at[pl.ds(base, K)], idr.at[j], sem_i[j])

        def idx_wait(j):
            pltpu.make_async_copy(dst1.at[pl.ds(0, K)], idr.at[j],
                                  sem_i[j]).wait()

        def s_desc(j):
            return pltpu.make_async_copy(ones, acc.at[idr.at[j]], sem_s[j])

        idx_start(0, 0)
        idx_start(1, 1)
        pltpu.make_async_copy(ones_hbm, ones, sem_o).wait()

        @pl.when(s == 0)
        def _():
            pltpu.make_async_copy(zeros, acc, sem_z).wait()

        plsc.subcore_barrier()

        def stage(q, j):
            n = 4 * q + j

            @pl.when(n >= 2)
            def _():
                s_desc((j + 2) % 4).wait()

            idx_wait(j)

            @pl.when(n + 2 < nc_chunks)
            def _():
                idx_start(n + 2, (j + 2) % 4)

            pltpu.async_copy(ones, acc.at[idr.at[j]], sem_s[j], add=True)

        @pl.loop(0, nc_chunks // 4)
        def _(q):
            for j in range(4):
                stage(q, j)

        s_desc(2).wait()
        s_desc(3).wait()
        plsc.subcore_barrier()
        _copy_out_stripes(s, acc, out, c, n_out)

    return pl.kernel(
        body,
        out_type=jax.ShapeDtypeStruct((NC, n_out, HH), jnp.float32),
        mesh=_sc_mesh(),
        scratch_types=[
            pltpu.VMEM_SHARED((n_acc, HH), jnp.float32),
            pltpu.VMEM((K, HH), jnp.float32),
            pltpu.VMEM((4, K), jnp.int32),
        ] + [pltpu.SemaphoreType.DMA] * 10,
    )


def _t_pre(x, W1, deg2):
    def body(x_ref, w_ref, d_ref, zs_ref, inv_ref):
        deg = d_ref[0, :, 0:1] + d_ref[1, :, 0:1] + 1.0
        inv = lax.rsqrt(deg)
        z = jnp.dot(x_ref[...], w_ref[...], preferred_element_type=jnp.float32)
        zs = z * inv
        zs_ref[0] = zs[:, :HH]
        zs_ref[1] = zs[:, HH:]
        inv_ref[...] = inv

    return pl.pallas_call(
        body,
        grid=(N // BN,),
        in_specs=[
            pl.BlockSpec((BN, F_IN), lambda i: (i, 0)),
            pl.BlockSpec((F_IN, H), lambda i: (0, 0)),
            pl.BlockSpec((NC, BN, HH), lambda i: (0, i, 0)),
        ],
        out_specs=[
            pl.BlockSpec((NC, BN, HH), lambda i: (0, i, 0)),
            pl.BlockSpec((BN, 1), lambda i: (i, 0)),
        ],
        out_shape=[
            jax.ShapeDtypeStruct((NC, N, HH), jnp.float32),
            jax.ShapeDtypeStruct((N, 1), jnp.float32),
        ],
    )(x, W1, deg2)


def _t_mid(s2, zs2, inv, b_row, W):
    def body(s_ref, zs_ref, inv_ref, b_ref, w_ref, o_ref):
        agg = jnp.concatenate(
            [s_ref[0] + zs_ref[0], s_ref[1] + zs_ref[1]], axis=1)
        inv = inv_ref[...]
        h = agg * inv + b_ref[...]
        h = jnp.where(h > 0, h, 0.01 * h)
        z = jnp.dot(h, w_ref[...], preferred_element_type=jnp.float32)
        zn = z * inv
        o_ref[0] = zn[:, :HH]
        o_ref[1] = zn[:, HH:]

    return pl.pallas_call(
        body,
        grid=(N // BN,),
        in_specs=[
            pl.BlockSpec((NC, BN, HH), lambda i: (0, i, 0)),
            pl.BlockSpec((NC, BN, HH), lambda i: (0, i, 0)),
            pl.BlockSpec((BN, 1), lambda i: (i, 0)),
            pl.BlockSpec((1, H), lambda i: (0, 0)),
            pl.BlockSpec((H, H), lambda i: (0, 0)),
        ],
        out_specs=pl.BlockSpec((NC, BN, HH), lambda i: (0, i, 0)),
        out_shape=jax.ShapeDtypeStruct((NC, N, HH), jnp.float32),
    )(s2, zs2, inv, b_row, W)


def _t_post(s3, zs3, inv, b_row):
    def body(s_ref, zs_ref, inv_ref, b_ref, o_ref):
        agg = jnp.concatenate(
            [s_ref[0] + zs_ref[0], s_ref[1] + zs_ref[1]], axis=1)
        h = agg * inv_ref[...] + b_ref[...]
        o_ref[0] = h[:, :HH]
        o_ref[1] = h[:, HH:]

    return pl.pallas_call(
        body,
        grid=(N // BN,),
        in_specs=[
            pl.BlockSpec((NC, BN, HH), lambda i: (0, i, 0)),
            pl.BlockSpec((NC, BN, HH), lambda i: (0, i, 0)),
            pl.BlockSpec((BN, 1), lambda i: (i, 0)),
            pl.BlockSpec((1, H), lambda i: (0, 0)),
        ],
        out_specs=pl.BlockSpec((NC, BN, HH), lambda i: (0, i, 0)),
        out_shape=jax.ShapeDtypeStruct((NC, N, HH), jnp.float32),
    )(s3, zs3, inv, b_row)


def _t_head(psum, cnt, Wlin, blin_row, Wout, bout_row):
    def body(ps_ref, c_ref, wl_ref, bl_ref, wo_ref, bo_ref,
             logits_ref, probs_ref, emb_ref, el_ref):
        sum_pool = jnp.concatenate([ps_ref[0], ps_ref[1]], axis=1)
        counts = c_ref[0, :, 0:1] + c_ref[1, :, 0:1]
        mean_pool = sum_pool / jnp.maximum(counts, 1.0)
        embeds = jnp.concatenate([sum_pool, mean_pool], axis=1)
        hl = jnp.dot(embeds, wl_ref[...],
                     preferred_element_type=jnp.float32) + bl_ref[...]
        el = jnp.maximum(hl, 0.0)
        logits = jnp.dot(el, wo_ref[...],
                         preferred_element_type=jnp.float32) + bo_ref[...]
        m = jnp.max(logits, axis=1, keepdims=True)
        ex = jnp.exp(logits - m)
        probs = ex / jnp.sum(ex, axis=1, keepdims=True)
        logits_ref[...] = logits
        probs_ref[...] = probs
        emb_ref[...] = embeds
        el_ref[...] = el

    return pl.pallas_call(
        body,
        out_shape=[
            jax.ShapeDtypeStruct((G, C), jnp.float32),
            jax.ShapeDtypeStruct((G, C), jnp.float32),
            jax.ShapeDtypeStruct((G, 2 * H), jnp.float32),
            jax.ShapeDtypeStruct((G, H), jnp.float32),
        ],
    )(psum, cnt, Wlin, blin_row, Wout, bout_row)


def _pad_reshape(idx, total, fill, shape, spread=PAD):
    npad = total - idx.shape[0]
    pad = fill + jnp.arange(npad, dtype=jnp.int32) % spread
    return jnp.concatenate([idx, pad]).reshape(shape)


def kernel(x, edge_index, batch, W1, b1, W2, b2, W3, b3, Wlin, blin, Wout, bout):
    src = edge_index[0]
    dst = edge_index[1]

    src_e = _pad_reshape(src, E_PAD, 0, (NS, EDGE_CHUNKS, K), spread=N)
    dst_e = _pad_reshape(dst, E_PAD, N, (NS, EDGE_CHUNKS, K))
    dst_h = _pad_reshape(dst, E_PAD32, N, (NC, NS, EDGE_CHUNKS32, K))
    node_p = _pad_reshape(jnp.arange(N, dtype=jnp.int32), N_PAD16, 0,
                          (NS, NODE_CHUNKS, K), spread=N)
    batch_p = _pad_reshape(batch, N_PAD16, G, (NS, NODE_CHUNKS, K))
    batch_h = _pad_reshape(batch, N_PAD32, G, (NC, NS, NODE_CHUNKS32, K))

    z_nodes = jnp.zeros((N + PAD, HH), jnp.float32)
    z_pool = jnp.zeros((G + PAD, HH), jnp.float32)

    gs_edge = _gs_pass(EDGE_CHUNKS, N, N + PAD, N)
    gs_pool = _gs_pass(NODE_CHUNKS, N, G + PAD, G)
    hist_deg = _hist_pass(EDGE_CHUNKS32, N + PAD, N)
    hist_cnt = _hist_pass(NODE_CHUNKS32, G + PAD, G)

    ones_k = jnp.ones((K, HH), jnp.float32)

    deg2 = hist_deg(dst_h, ones_k, z_nodes)
    zs1, inv = _t_pre(x, W1, deg2)
    s1 = gs_edge(zs1, src_e, dst_e, z_nodes)
    zs2 = _t_mid(s1, zs1, inv, b1.reshape(1, H), W2)
    s2 = gs_edge(zs2, src_e, dst_e, z_nodes)
    zs3 = _t_mid(s2, zs2, inv, b2.reshape(1, H), W3)
    s3 = gs_edge(zs3, src_e, dst_e, z_nodes)
    h3 = _t_post(s3, zs3, inv, b3.reshape(1, H))
    psum = gs_pool(h3, node_p, batch_p, z_pool)
    z_cnt = z_pool + psum[0, 0, 0] * 0.0
    cnt = hist_cnt(batch_h, ones_k, z_cnt)
    logits, probs, embeds, embeds_last = _t_head(
        psum, cnt, Wlin, blin.reshape(1, H), Wout, bout.reshape(1, C))
    return (logits, probs, embeds, embeds_last)


# speedup vs baseline: 1.6866x; 1.0207x over previous
import functools

import jax
import jax.numpy as jnp
from jax import lax
from jax.experimental import pallas as pl
from jax.experimental.pallas import tpu as pltpu
from jax.experimental.pallas import tpu_sc as plsc

N = 10000
E = 320000
F_IN = 128
H = 256
C = 32
G = 64

NC = 2
NS = 16
K = 128
HH = H // 2
PAD = 128

IB = 8


def _round8(x):
    return -(-x // IB) * IB


EDGE_CHUNKS = _round8(-(-E // (NS * K)))
E_PAD = EDGE_CHUNKS * NS * K
NODE_CHUNKS = _round8(-(-N // (NS * K)))
N_PAD16 = NODE_CHUNKS * NS * K
EDGE_CHUNKS32 = _round8(-(-E // (NC * NS * K)))
E_PAD32 = EDGE_CHUNKS32 * NC * NS * K
NODE_CHUNKS32 = _round8(-(-N // (NC * NS * K)))
N_PAD32 = NODE_CHUNKS32 * NC * NS * K

BN = 1000


def _sc_mesh():
    return plsc.VectorSubcoreMesh(core_axis_name="c", subcore_axis_name="s")


def _copy_out_stripes(s, acc, out, c, n_out):
    if n_out % (NS * 8) == 0:
        rows = n_out // NS

        pltpu.sync_copy(acc.at[pl.ds(s * rows, rows)],
                        out.at[c].at[pl.ds(s * rows, rows)])
    elif n_out >= NS * 8:
        base = (n_out // NS) & ~7
        rem = n_out - base * NS

        @pl.when(s < NS - 1)
        def _():
            pltpu.sync_copy(acc.at[pl.ds(s * base, base)],
                            out.at[c].at[pl.ds(s * base, base)])

        @pl.when(s == NS - 1)
        def _():
            pltpu.sync_copy(acc.at[pl.ds((NS - 1) * base, base + rem)],
                            out.at[c].at[pl.ds((NS - 1) * base, base + rem)])
    else:
        @pl.when(s == 0)
        def _():
            pltpu.sync_copy(acc.at[pl.ds(0, n_out)], out.at[c])


@functools.cache
def _gs_pass(nc_chunks, n_tbl, n_acc, n_out):
    assert nc_chunks % IB == 0

    def body(tbl, src3, dst3, zeros, out, acc, idx_s, idx_d, rows):
        c = lax.axis_index("c")
        s = lax.axis_index("s")

        @pl.when(s == 0)
        def _():
            pltpu.sync_copy(zeros, acc)

        plsc.subcore_barrier()

        @pl.loop(0, nc_chunks // IB)
        def _(b):
            pltpu.sync_copy(src3.at[s, pl.ds(b * IB, IB)], idx_s)
            pltpu.sync_copy(dst3.at[s, pl.ds(b * IB, IB)], idx_d)
            for j in range(IB):
                pltpu.sync_copy(tbl.at[c].at[idx_s.at[j]], rows)
                pltpu.sync_copy(rows, acc.at[idx_d.at[j]], add=True)

        plsc.subcore_barrier()
        _copy_out_stripes(s, acc, out, c, n_out)

    return pl.kernel(
        body,
        out_type=jax.ShapeDtypeStruct((NC, n_out, HH), jnp.float32),
        mesh=_sc_mesh(),
        scratch_types=[
            pltpu.VMEM_SHARED((n_acc, HH), jnp.float32),
            pltpu.VMEM((IB, K), jnp.int32),
            pltpu.VMEM((IB, K), jnp.int32),
            pltpu.VMEM((K, HH), jnp.float32),
        ],
    )


@functools.cache
def _gs_pass2(nc_chunks, n_tbl, n_acc, n_out, scat2):
    assert nc_chunks % IB == 0

    def body(tbl, src3, dst3, zeros, out, acc, idx_s, idx_d, rows):
        c = lax.axis_index("c")
        s = lax.axis_index("s")

        @pl.when(s == 0)
        def _():
            pltpu.sync_copy(zeros, acc)

        plsc.subcore_barrier()

        @pl.loop(0, nc_chunks // IB)
        def _(b):
            pltpu.sync_copy(src3.at[s, pl.ds(b * IB, IB)], idx_s)
            pltpu.sync_copy(dst3.at[s, pl.ds(b * IB, IB)], idx_d)
            for j in range(IB // 2):
                pltpu.sync_copy(tbl.at[c].at[idx_s.at[pl.ds(2 * j, 2)]],
                                rows)
                if scat2:
                    pltpu.sync_copy(rows,
                                    acc.at[idx_d.at[pl.ds(2 * j, 2)]],
                                    add=True)
                else:
                    pltpu.sync_copy(rows.at[0],
                                    acc.at[idx_d.at[2 * j]], add=True)
                    pltpu.sync_copy(rows.at[1],
                                    acc.at[idx_d.at[2 * j + 1]], add=True)

        plsc.subcore_barrier()
        _copy_out_stripes(s, acc, out, c, n_out)

    return pl.kernel(
        body,
        out_type=jax.ShapeDtypeStruct((NC, n_out, HH), jnp.float32),
        mesh=_sc_mesh(),
        scratch_types=[
            pltpu.VMEM_SHARED((n_acc, HH), jnp.float32),
            pltpu.VMEM((IB, K), jnp.int32),
            pltpu.VMEM((IB, K), jnp.int32),
            pltpu.VMEM((2, K, HH), jnp.float32),
        ],
    )


@functools.cache
def _gs_pipe(nc_chunks, n_tbl, n_acc, n_out):
    assert nc_chunks % 4 == 0

    def body(tbl, src1, dst1, zeros, out, acc, rows2, isr, idr, sem_z,
             sg0, sg1, ss0, ss1, sis0, sis1, sis2, sis3,
             sid0, sid1, sid2, sid3):
        sem_g = [sg0, sg1]
        sem_s = [ss0, ss1]
        sem_is = [sis0, sis1, sis2, sis3]
        sem_id = [sid0, sid1, sid2, sid3]
        c = lax.axis_index("c")
        s = lax.axis_index("s")

        @pl.when(s == 0)
        def _():
            pltpu.async_copy(zeros, acc, sem_z)

        def idx_start(n, j):
            base = (s * nc_chunks + n) * K
            pltpu.async_copy(src1.at[pl.ds(base, K)], isr.at[j], sem_is[j])
            pltpu.async_copy(dst1.at[pl.ds(base, K)], idr.at[j], sem_id[j])

        def idx_wait(j):
            pltpu.make_async_copy(src1.at[pl.ds(0, K)], isr.at[j],
                                  sem_is[j]).wait()
            pltpu.make_async_copy(dst1.at[pl.ds(0, K)], idr.at[j],
                                  sem_id[j]).wait()

        def g_desc(j, a):
            return pltpu.make_async_copy(tbl.at[c].at[isr.at[j]],
                                         rows2.at[a], sem_g[a])

        def s_desc(j, a):
            return pltpu.make_async_copy(rows2.at[a], acc.at[idr.at[j]],
                                         sem_s[a])

        idx_start(0, 0)
        idx_start(1, 1)

        @pl.when(s == 0)
        def _():
            pltpu.make_async_copy(zeros, acc, sem_z).wait()

        plsc.subcore_barrier()

        def stage(q, j):
            n = 4 * q + j
            a = j % 2

            @pl.when(n >= 2)
            def _():
                s_desc((j + 2) % 4, a).wait()

            idx_wait(j)
            g_desc(j, a).start()

            @pl.when(n + 2 < nc_chunks)
            def _():
                idx_start(n + 2, (j + 2) % 4)

            g_desc(j, a).wait()
            pltpu.async_copy(rows2.at[a], acc.at[idr.at[j]], sem_s[a],
                             add=True)

        @pl.loop(0, nc_chunks // 4)
        def _(q):
            for j in range(4):
                stage(q, j)

        s_desc(2, 0).wait()
        s_desc(3, 1).wait()
        plsc.subcore_barrier()
        _copy_out_stripes(s, acc, out, c, n_out)

    return pl.kernel(
        body,
        out_type=jax.ShapeDtypeStruct((NC, n_out, HH), jnp.float32),
        mesh=_sc_mesh(),
        scratch_types=[
            pltpu.VMEM_SHARED((n_acc, HH), jnp.float32),
            pltpu.VMEM((2, K, HH), jnp.float32),
            pltpu.VMEM((4, K), jnp.int32),
            pltpu.VMEM((4, K), jnp.int32),
        ] + [pltpu.SemaphoreType.DMA] * 13,
    )


@functools.cache
def _hist_pass(nc_chunks, n_acc, n_out):
    assert nc_chunks % IB == 0

    def body(dst4, ones_hbm, zeros, out, acc, idx_d, ones):
        c = lax.axis_index("c")
        s = lax.axis_index("s")

        pltpu.sync_copy(ones_hbm, ones)

        @pl.when(s == 0)
        def _():
            pltpu.sync_copy(zeros, acc)

        plsc.subcore_barrier()

        @pl.loop(0, nc_chunks // IB)
        def _(b):
            pltpu.sync_copy(dst4.at[c, s, pl.ds(b * IB, IB)], idx_d)
            for j in range(IB):
                pltpu.sync_copy(ones, acc.at[idx_d.at[j]], add=True)

        plsc.subcore_barrier()
        _copy_out_stripes(s, acc, out, c, n_out)

    return pl.kernel(
        body,
        out_type=jax.ShapeDtypeStruct((NC, n_out, HH), jnp.float32),
        mesh=_sc_mesh(),
        scratch_types=[
            pltpu.VMEM_SHARED((n_acc, HH), jnp.float32),
            pltpu.VMEM((IB, K), jnp.int32),
            pltpu.VMEM((K, HH), jnp.float32),
        ],
    )


@functools.cache
def _hist_pipe(nc_chunks, n_acc, n_out):
    assert nc_chunks % 4 == 0

    def body(dst1, ones_hbm, zeros, out, acc, ones, idr,
             sem_z, sem_o, ss0, ss1, ss2, ss3, si0, si1, si2, si3):
        sem_s = [ss0, ss1, ss2, ss3]
        sem_i = [si0, si1, si2, si3]
        c = lax.axis_index("c")
        s = lax.axis_index("s")

        @pl.when(s == 0)
        def _():
            pltpu.async_copy(zeros, acc, sem_z)

        pltpu.async_copy(ones_hbm, ones, sem_o)

        def idx_start(n, j):
            base = ((c * NS + s) * nc_chunks + n) * K
            pltpu.async_copy(dst1.at[pl.ds(base, K)], idr.at[j], sem_i[j])

        def idx_wait(j):
            pltpu.make_async_copy(dst1.at[pl.ds(0, K)], idr.at[j],
                                  sem_i[j]).wait()

        def s_desc(j):
            return pltpu.make_async_copy(ones, acc.at[idr.at[j]], sem_s[j])

        idx_start(0, 0)
        idx_start(1, 1)
        pltpu.make_async_copy(ones_hbm, ones, sem_o).wait()

        @pl.when(s == 0)
        def _():
            pltpu.make_async_copy(zeros, acc, sem_z).wait()

        plsc.subcore_barrier()

        def stage(q, j):
            n = 4 * q + j

            @pl.when(n >= 2)
            def _():
                s_desc((j + 2) % 4).wait()

            idx_wait(j)

            @pl.when(n + 2 < nc_chunks)
            def _():
                idx_start(n + 2, (j + 2) % 4)

            pltpu.async_copy(ones, acc.at[idr.at[j]], sem_s[j], add=True)

        @pl.loop(0, nc_chunks // 4)
        def _(q):
            for j in range(4):
                stage(q, j)

        s_desc(2).wait()
        s_desc(3).wait()
        plsc.subcore_barrier()
        _copy_out_stripes(s, acc, out, c, n_out)

    return pl.kernel(
        body,
        out_type=jax.ShapeDtypeStruct((NC, n_out, HH), jnp.float32),
        mesh=_sc_mesh(),
        scratch_types=[
            pltpu.VMEM_SHARED((n_acc, HH), jnp.float32),
            pltpu.VMEM((K, HH), jnp.float32),
            pltpu.VMEM((4, K), jnp.int32),
        ] + [pltpu.SemaphoreType.DMA] * 10,
    )


def _t_pre(x, W1, deg2):
    def body(x_ref, w_ref, d_ref, zs_ref, inv_ref):
        deg = d_ref[0, :, 0:1] + d_ref[1, :, 0:1] + 1.0
        inv = lax.rsqrt(deg)
        z = jnp.dot(x_ref[...], w_ref[...], preferred_element_type=jnp.float32)
        zs = z * inv
        zs_ref[0] = zs[:, :HH]
        zs_ref[1] = zs[:, HH:]
        inv_ref[...] = inv

    return pl.pallas_call(
        body,
        grid=(N // BN,),
        in_specs=[
            pl.BlockSpec((BN, F_IN), lambda i: (i, 0)),
            pl.BlockSpec((F_IN, H), lambda i: (0, 0)),
            pl.BlockSpec((NC, BN, HH), lambda i: (0, i, 0)),
        ],
        out_specs=[
            pl.BlockSpec((NC, BN, HH), lambda i: (0, i, 0)),
            pl.BlockSpec((BN, 1), lambda i: (i, 0)),
        ],
        out_shape=[
            jax.ShapeDtypeStruct((NC, N, HH), jnp.float32),
            jax.ShapeDtypeStruct((N, 1), jnp.float32),
        ],
    )(x, W1, deg2)


def _t_mid(s2, zs2, inv, b_row, W):
    def body(s_ref, zs_ref, inv_ref, b_ref, w_ref, o_ref):
        agg = jnp.concatenate(
            [s_ref[0] + zs_ref[0], s_ref[1] + zs_ref[1]], axis=1)
        inv = inv_ref[...]
        h = agg * inv + b_ref[...]
        h = jnp.where(h > 0, h, 0.01 * h)
        z = jnp.dot(h, w_ref[...], preferred_element_type=jnp.float32)
        zn = z * inv
        o_ref[0] = zn[:, :HH]
        o_ref[1] = zn[:, HH:]

    return pl.pallas_call(
        body,
        grid=(N // BN,),
        in_specs=[
            pl.BlockSpec((NC, BN, HH), lambda i: (0, i, 0)),
            pl.BlockSpec((NC, BN, HH), lambda i: (0, i, 0)),
            pl.BlockSpec((BN, 1), lambda i: (i, 0)),
            pl.BlockSpec((1, H), lambda i: (0, 0)),
            pl.BlockSpec((H, H), lambda i: (0, 0)),
        ],
        out_specs=pl.BlockSpec((NC, BN, HH), lambda i: (0, i, 0)),
        out_shape=jax.ShapeDtypeStruct((NC, N, HH), jnp.float32),
    )(s2, zs2, inv, b_row, W)


def _t_post(s3, zs3, inv, b_row):
    def body(s_ref, zs_ref, inv_ref, b_ref, o_ref):
        agg = jnp.concatenate(
            [s_ref[0] + zs_ref[0], s_ref[1] + zs_ref[1]], axis=1)
        h = agg * inv_ref[...] + b_ref[...]
        o_ref[0] = h[:, :HH]
        o_ref[1] = h[:, HH:]

    return pl.pallas_call(
        body,
        grid=(N // BN,),
        in_specs=[
            pl.BlockSpec((NC, BN, HH), lambda i: (0, i, 0)),
            pl.BlockSpec((NC, BN, HH), lambda i: (0, i, 0)),
            pl.BlockSpec((BN, 1), lambda i: (i, 0)),
            pl.BlockSpec((1, H), lambda i: (0, 0)),
        ],
        out_specs=pl.BlockSpec((NC, BN, HH), lambda i: (0, i, 0)),
        out_shape=jax.ShapeDtypeStruct((NC, N, HH), jnp.float32),
    )(s3, zs3, inv, b_row)


def _t_head(h3, batch2d, Wlin, blin_row, Wout, bout_row):
    def body(h_ref, b_ref, wl_ref, bl_ref, wo_ref, bo_ref,
             logits_ref, probs_ref, emb_ref, el_ref):
        h = jnp.concatenate([h_ref[0], h_ref[1]], axis=1)
        seg = jax.lax.broadcasted_iota(jnp.int32, (N, G), 1)
        onehot = (b_ref[...] == seg).astype(jnp.float32)
        dims = (((0,), (0,)), ((), ()))
        sum_pool = jax.lax.dot_general(
            onehot, h, dims, precision=jax.lax.Precision.HIGHEST,
            preferred_element_type=jnp.float32)
        counts = jax.lax.dot_general(
            onehot, jnp.ones((N, 1), jnp.float32), dims,
            preferred_element_type=jnp.float32)
        mean_pool = sum_pool / jnp.maximum(counts, 1.0)
        embeds = jnp.concatenate([sum_pool, mean_pool], axis=1)
        hl = jnp.dot(embeds, wl_ref[...],
                     preferred_element_type=jnp.float32) + bl_ref[...]
        el = jnp.maximum(hl, 0.0)
        logits = jnp.dot(el, wo_ref[...],
                         preferred_element_type=jnp.float32) + bo_ref[...]
        m = jnp.max(logits, axis=1, keepdims=True)
        ex = jnp.exp(logits - m)
        probs = ex / jnp.sum(ex, axis=1, keepdims=True)
        logits_ref[...] = logits
        probs_ref[...] = probs
        emb_ref[...] = embeds
        el_ref[...] = el

    return pl.pallas_call(
        body,
        out_shape=[
            jax.ShapeDtypeStruct((G, C), jnp.float32),
            jax.ShapeDtypeStruct((G, C), jnp.float32),
            jax.ShapeDtypeStruct((G, 2 * H), jnp.float32),
            jax.ShapeDtypeStruct((G, H), jnp.float32),
        ],
    )(h3, batch2d, Wlin, blin_row, Wout, bout_row)


def _pad_reshape(idx, total, fill, shape, spread=PAD):
    npad = total - idx.shape[0]
    pad = fill + jnp.arange(npad, dtype=jnp.int32) % spread
    return jnp.concatenate([idx, pad]).reshape(shape)


def kernel(x, edge_index, batch, W1, b1, W2, b2, W3, b3, Wlin, blin, Wout, bout):
    src = edge_index[0]
    dst = edge_index[1]

    src_e = _pad_reshape(src, E_PAD, 0, (NS, EDGE_CHUNKS, K), spread=N)
    dst_e = _pad_reshape(dst, E_PAD, N, (NS, EDGE_CHUNKS, K))
    dst_h = _pad_reshape(dst, E_PAD32, N, (NC, NS, EDGE_CHUNKS32, K))

    z_nodes = jnp.zeros((N + PAD, HH), jnp.float32)

    gs_edge = _gs_pass(EDGE_CHUNKS, N, N + PAD, N)
    hist_deg = _hist_pass(EDGE_CHUNKS32, N + PAD, N)

    ones_k = jnp.ones((K, HH), jnp.float32)

    deg2 = hist_deg(dst_h, ones_k, z_nodes)
    zs1, inv = _t_pre(x, W1, deg2)
    s1 = gs_edge(zs1, src_e, dst_e, z_nodes)
    zs2 = _t_mid(s1, zs1, inv, b1.reshape(1, H), W2)
    s2 = gs_edge(zs2, src_e, dst_e, z_nodes)
    zs3 = _t_mid(s2, zs2, inv, b2.reshape(1, H), W3)
    s3 = gs_edge(zs3, src_e, dst_e, z_nodes)
    h3 = _t_post(s3, zs3, inv, b3.reshape(1, H))
    logits, probs, embeds, embeds_last = _t_head(
        h3, batch.reshape(N, 1), Wlin, blin.reshape(1, H), Wout,
        bout.reshape(1, C))
    return (logits, probs, embeds, embeds_last)
